# baseline (speedup 1.0000x reference)
import functools

import numpy as np

import jax
import jax.numpy as jnp
from jax import lax
from jax.experimental import pallas as pl
from jax.experimental.pallas import tpu as pltpu
from jax.experimental.pallas import tpu_sc as plsc

EMBED = 512
HEADS = 8
LEVELS = 4
POINTS = 4
DH = EMBED // HEADS
HLP = HEADS * LEVELS * POINTS


def _mm_bias_kernel(x_ref, w_ref, b_ref, o_ref):
    o_ref[...] = (
        jnp.dot(x_ref[...], w_ref[...], preferred_element_type=jnp.float32)
        + b_ref[...]
    ).astype(o_ref.dtype)


def _mm_pack_kernel(x_ref, w_ref, b_ref, o_ref):
    o = (jnp.dot(x_ref[...], w_ref[...], preferred_element_type=jnp.float32)
         + b_ref[...])
    h = o.shape[1] // 2
    lo = jax.lax.bitcast_convert_type(
        o[:, :h].astype(jnp.bfloat16), jnp.uint16).astype(jnp.int32)
    hi = jax.lax.bitcast_convert_type(
        o[:, h:].astype(jnp.bfloat16), jnp.uint16).astype(jnp.int32)
    o_ref[...] = (hi << 16) | lo


def _matmul_pack(x, w, b, blk):
    m, k = x.shape
    n = w.shape[1]
    return pl.pallas_call(
        _mm_pack_kernel,
        grid=(m // blk,),
        in_specs=[
            pl.BlockSpec((blk, k), lambda i: (i, 0)),
            pl.BlockSpec((k, n), lambda i: (0, 0)),
            pl.BlockSpec((1, n), lambda i: (0, 0)),
        ],
        out_specs=pl.BlockSpec((blk, n // 2), lambda i: (i, 0)),
        out_shape=jax.ShapeDtypeStruct((m, n // 2), jnp.int32),
    )(x, w, b.reshape(1, n))


def _matmul_bias(x, w, b, blk, out_dtype=jnp.float32):
    m, k = x.shape
    n = w.shape[1]
    return pl.pallas_call(
        _mm_bias_kernel,
        grid=(m // blk,),
        in_specs=[
            pl.BlockSpec((blk, k), lambda i: (i, 0)),
            pl.BlockSpec((k, n), lambda i: (0, 0)),
            pl.BlockSpec((1, n), lambda i: (0, 0)),
        ],
        out_specs=pl.BlockSpec((blk, n), lambda i: (i, 0)),
        out_shape=jax.ShapeDtypeStruct((m, n), out_dtype),
    )(x, w, b.reshape(1, n))


def _taps_kernel(q_ref, woff_ref, boff_ref, wattn_ref, battn_ref, g_ref,
                 refx_ref, refy_ref, fconst_ref, iconst_ref,
                 p0_ref, p1_ref, p2_ref, p3_ref):
    q = q_ref[0]
    off2 = jnp.dot(q, woff_ref[...], preferred_element_type=jnp.float32) + boff_ref[...]
    z = jnp.dot(q, wattn_ref[...], preferred_element_type=jnp.float32) + battn_ref[...]
    e = jnp.exp(z)
    s = jnp.dot(e, g_ref[...], preferred_element_type=jnp.float32)
    aw = e / s

    invw = fconst_ref[0:1]
    invh = fconst_ref[1:2]
    wf = fconst_ref[2:3]
    hf = fconst_ref[3:4]
    wi = iconst_ref[0:1]
    hi = iconst_ref[1:2]
    n = pl.program_id(0)
    basev = iconst_ref[pl.ds(2 + n, 1)]

    px = (refx_ref[0] + off2[:, :HLP] * invw) * wf - 0.5
    py = (refy_ref[0] + off2[:, HLP:] * invh) * hf - 0.5
    x0f = jnp.floor(px)
    y0f = jnp.floor(py)
    lx = px - x0f
    ly = py - y0f
    x0 = x0f.astype(jnp.int32)
    y0 = y0f.astype(jnp.int32)

    outs = ((p0_ref, 0, 0, (1.0 - lx) * (1.0 - ly)),
            (p1_ref, 0, 1, lx * (1.0 - ly)),
            (p2_ref, 1, 0, (1.0 - lx) * ly),
            (p3_ref, 1, 1, lx * ly))
    for p_ref, dy, dx, bl in outs:
        xi = x0 + dx
        yi = y0 + dy
        valid = (xi >= 0) & (xi < wi) & (yi >= 0) & (yi < hi)
        xc = jnp.clip(xi, 0, wi - 1)
        yc = jnp.clip(yi, 0, hi - 1)
        idx = basev + (yc * wi + xc) * HEADS
        wt = aw * bl * valid.astype(jnp.float32)
        q = jnp.minimum((wt * 32768.0).astype(jnp.int32), 32767)
        p_ref[0] = (idx << 15) | q


def _compute_taps(q3, woffp, boffp, wattn, battn, g, refx, refy, fconst, iconst, bq):
    n, lq, _ = q3.shape
    grid = (n, lq // bq)
    row_specs = pl.BlockSpec((1, bq, HLP), lambda i, j: (i, j, 0))
    out_sd = jax.ShapeDtypeStruct((n, lq, HLP), jnp.int32)
    return pl.pallas_call(
        _taps_kernel,
        grid=grid,
        in_specs=[
            pl.BlockSpec((1, bq, EMBED), lambda i, j: (i, j, 0)),
            pl.BlockSpec((EMBED, 2 * HLP), lambda i, j: (0, 0)),
            pl.BlockSpec((1, 2 * HLP), lambda i, j: (0, 0)),
            pl.BlockSpec((EMBED, HLP), lambda i, j: (0, 0)),
            pl.BlockSpec((1, HLP), lambda i, j: (0, 0)),
            pl.BlockSpec((HLP, HLP), lambda i, j: (0, 0)),
            row_specs,
            row_specs,
            pl.BlockSpec((8, HLP), lambda i, j: (0, 0)),
            pl.BlockSpec((8, HLP), lambda i, j: (0, 0)),
        ],
        out_specs=(row_specs,) * 4,
        out_shape=(out_sd,) * 4,
    )(q3, woffp, boffp.reshape(1, 2 * HLP), wattn, battn.reshape(1, HLP),
      g, refx, refy, fconst, iconst)


SC_CORES = 2
SC_SUBCORES = 16


def _make_sc_gather(s_total, table_rows):
    nw = SC_CORES * SC_SUBCORES
    chunk = 16
    per_w = s_total // nw
    n_chunks = per_w // chunk
    kb = chunk * LEVELS * POINTS
    kh = kb // 2

    mesh = plsc.VectorSubcoreMesh(core_axis_name="c", subcore_axis_name="s",
                                  num_cores=SC_CORES, num_subcores=SC_SUBCORES)

    scratch = (
        [pltpu.VMEM((kb,), jnp.int32) for _ in range(8)]
        + [pltpu.VMEM((kb,), jnp.int32) for _ in range(8)]
        + [pltpu.VMEM((kb, DH // 2), jnp.int32) for _ in range(8)]
        + [pltpu.VMEM((chunk * DH,), jnp.float32) for _ in range(2)]
        + [pltpu.SemaphoreType.DMA for _ in range(6)]
    )

    @functools.partial(
        pl.kernel,
        out_type=jax.ShapeDtypeStruct((s_total * DH,), jnp.float32),
        mesh=mesh,
        scratch_types=scratch,
        compiler_params=pltpu.CompilerParams(use_tc_tiling_on_sc=False,
                                             needs_layout_passes=False),
    )
    def sc_gather(table, p0, p1, p2, p3, out, *scr):
        pv = (scr[0:4], scr[4:8])
        iv = (scr[8:12], scr[12:16])
        rv = (scr[16:20], scr[20:24])
        ov = scr[24:26]
        sem_i = scr[26:28]
        sem_r = scr[28:30]
        sem_o = scr[30:32]
        pk_hbm = (p0, p1, p2, p3)
        wid = lax.axis_index("s") * SC_CORES + lax.axis_index("c")
        w_base = wid * per_w
        lp = LEVELS * POINTS

        def fire_idx(c, s):
            fb = (w_base + c * chunk) * lp
            for t in range(4):
                pltpu.async_copy(pk_hbm[t].at[pl.ds(fb, kb)], pv[s][t], sem_i[s])

        def drain_idx(s):
            for t in range(4):
                pltpu.make_async_copy(pk_hbm[t].at[pl.ds(0, kb)], pv[s][t], sem_i[s]).wait()

        def unpack_idx(s):
            for t in range(4):
                for r in range(kb // 16):
                    v = pv[s][t][pl.ds(r * 16, 16)]
                    iv[s][t][pl.ds(r * 16, 16)] = lax.shift_right_logical(v, 15)

        def fire_rows(s):
            for t in range(4):
                for h in range(2):
                    pltpu.async_copy(table.at[iv[s][t].at[pl.ds(h * kh, kh)]],
                                     rv[s][t].at[pl.ds(h * kh, kh)], sem_r[s])

        def drain_rows(s):
            for t in range(4):
                for h in range(2):
                    pltpu.make_async_copy(
                        table.at[pl.ds(0, kh)],
                        rv[s][t].at[pl.ds(h * kh, kh)], sem_r[s]).wait()

        def fire_out(c, s):
            fb = (w_base + c * chunk) * DH
            pltpu.async_copy(ov[s], out.at[pl.ds(fb, chunk * DH)], sem_o[s])

        def drain_out(s):
            pltpu.make_async_copy(ov[s], out.at[pl.ds(0, chunk * DH)], sem_o[s]).wait()

        fire_idx(0, 0)
        fire_idx(1, 1)
        drain_idx(0)
        unpack_idx(0)
        fire_rows(0)

        def chunk_body(c, carry):
            for s in range(2):
                i = c * 2 + s
                drain_rows(s)

                @pl.when(i + 1 < n_chunks)
                def _():
                    drain_idx(1 - s)
                    unpack_idx(1 - s)
                    fire_rows(1 - s)

                @pl.when(i >= 2)
                def _():
                    drain_out(s)

                def compute_sample(k0):
                    accs = [jnp.zeros((16,), jnp.float32) for _ in range(DH // 16)]
                    for t in range(4):
                        pk = pv[s][t][pl.ds(k0 * lp, lp)]
                        wvec = (pk & 32767).astype(jnp.float32) * (1.0 / 32768.0)
                        for j in range(lp):
                            k = k0 * lp + j
                            wt = wvec[j]
                            for g in range(DH // 32):
                                iv32 = rv[s][t][k, pl.ds(g * 16, 16)]
                                ea = plsc.bitcast(iv32 << 16, jnp.float32)
                                eb = plsc.bitcast(iv32, jnp.float32)
                                accs[2 * g] = accs[2 * g] + wt * ea
                                accs[2 * g + 1] = accs[2 * g + 1] + wt * eb
                    for d in range(DH // 16):
                        ov[s][pl.ds(k0 * DH + d * 16, 16)] = accs[d]

                def sample_body(k2, carry2):
                    compute_sample(k2 * 2)
                    compute_sample(k2 * 2 + 1)
                    return carry2

                lax.fori_loop(0, chunk // 2, sample_body, 0)
                fire_out(i, s)

                @pl.when(i + 2 < n_chunks)
                def _():
                    fire_idx(i + 2, s)
            return carry

        lax.fori_loop(0, n_chunks // 2, chunk_body, 0)
        drain_out(0)
        drain_out(1)

    return sc_gather


def kernel(query, key, value, reference_points, spatial_shapes,
           level_start_index, W_off, b_off, W_attn, b_attn, W_val, b_val,
           W_out, b_out):
    lq, n, c = query.shape
    lin = value.shape[0]
    s_total = n * lq * HEADS

    q3 = query.transpose(1, 0, 2)
    v2d = value.transpose(1, 0, 2).reshape(n * lin, c)

    perm_v = np.concatenate([np.arange(HEADS)[:, None] * DH + np.arange(32)[None, :],
                             np.arange(HEADS)[:, None] * DH + 32 + np.arange(32)[None, :]]
                            ).reshape(2, HEADS * 32).reshape(-1)
    w_val2 = W_val[:, jnp.array(perm_v)]
    b_val2 = b_val[jnp.array(perm_v)]
    vproj = _matmul_pack(v2d, w_val2, b_val2, blk=640)
    table = vproj.reshape(n * lin * HEADS, DH // 2)

    wf = jnp.tile(jnp.repeat(spatial_shapes[:, 1], POINTS), HEADS).astype(jnp.float32)
    hf = jnp.tile(jnp.repeat(spatial_shapes[:, 0], POINTS), HEADS).astype(jnp.float32)
    fconst = jnp.zeros((8, HLP), jnp.float32)
    fconst = fconst.at[0].set(1.0 / wf).at[1].set(1.0 / hf).at[2].set(wf).at[3].set(hf)
    wi = jnp.tile(jnp.repeat(spatial_shapes[:, 1], POINTS), HEADS)
    hi = jnp.tile(jnp.repeat(spatial_shapes[:, 0], POINTS), HEADS)
    start_lane = jnp.tile(jnp.repeat(level_start_index, POINTS), HEADS)
    h_lane = jnp.repeat(jnp.arange(HEADS, dtype=jnp.int32), LEVELS * POINTS)
    iconst = jnp.zeros((8, HLP), jnp.int32)
    iconst = iconst.at[0].set(wi).at[1].set(hi)
    for nn in range(n):
        iconst = iconst.at[2 + nn].set((nn * lin + start_lane) * HEADS + h_lane)

    woffp = W_off.reshape(c, HLP, 2).transpose(0, 2, 1).reshape(c, 2 * HLP)
    boffp = b_off.reshape(HLP, 2).T.reshape(2 * HLP)

    g = jnp.kron(jnp.eye(HEADS * LEVELS, dtype=jnp.float32),
                 jnp.ones((POINTS, POINTS), jnp.float32))

    rp_x = reference_points[..., 0]
    rp_y = reference_points[..., 1]
    refx = jnp.broadcast_to(rp_x[:, :, None, :, None],
                            (n, lq, HEADS, LEVELS, POINTS)).reshape(n, lq, HLP)
    refy = jnp.broadcast_to(rp_y[:, :, None, :, None],
                            (n, lq, HEADS, LEVELS, POINTS)).reshape(n, lq, HLP)

    taps = _compute_taps(q3, woffp, boffp, W_attn, b_attn, g, refx, refy,
                         fconst, iconst, bq=320)
    pk_flat = [t.reshape(s_total * LEVELS * POINTS) for t in taps]

    sc = _make_sc_gather(s_total, table.shape[0])
    sampled = sc(table, *pk_flat)
    sampled2 = sampled.reshape(n * lq, HEADS * DH)

    order64 = np.concatenate([np.arange(0, 16), np.arange(32, 48),
                              np.arange(16, 32), np.arange(48, 64)])
    perm = (np.arange(HEADS)[:, None] * DH + order64[None, :]).reshape(-1)
    out = _matmul_bias(sampled2, W_out[jnp.array(perm)], b_out, blk=640)
    return out.reshape(n, lq, c).transpose(1, 0, 2)


# speedup vs baseline: 1.0038x; 1.0038x over previous
import functools

import numpy as np

import jax
import jax.numpy as jnp
from jax import lax
from jax.experimental import pallas as pl
from jax.experimental.pallas import tpu as pltpu
from jax.experimental.pallas import tpu_sc as plsc

EMBED = 512
HEADS = 8
LEVELS = 4
POINTS = 4
DH = EMBED // HEADS
HLP = HEADS * LEVELS * POINTS


def _mm_bias_kernel(x_ref, w_ref, b_ref, o_ref):
    o_ref[...] = (
        jnp.dot(x_ref[...], w_ref[...], preferred_element_type=jnp.float32)
        + b_ref[...]
    ).astype(o_ref.dtype)


def _mm_pack_kernel(x_ref, w_ref, b_ref, o_ref):
    o = (jnp.dot(x_ref[...], w_ref[...], preferred_element_type=jnp.float32)
         + b_ref[...])
    h = o.shape[1] // 2
    lo = jax.lax.bitcast_convert_type(
        o[:, :h].astype(jnp.bfloat16), jnp.uint16).astype(jnp.int32)
    hi = jax.lax.bitcast_convert_type(
        o[:, h:].astype(jnp.bfloat16), jnp.uint16).astype(jnp.int32)
    o_ref[...] = (hi << 16) | lo


def _matmul_pack(x, w, b, blk):
    m, k = x.shape
    n = w.shape[1]
    return pl.pallas_call(
        _mm_pack_kernel,
        grid=(m // blk,),
        in_specs=[
            pl.BlockSpec((blk, k), lambda i: (i, 0)),
            pl.BlockSpec((k, n), lambda i: (0, 0)),
            pl.BlockSpec((1, n), lambda i: (0, 0)),
        ],
        out_specs=pl.BlockSpec((blk, n // 2), lambda i: (i, 0)),
        out_shape=jax.ShapeDtypeStruct((m, n // 2), jnp.int32),
    )(x, w, b.reshape(1, n))


def _matmul_bias(x, w, b, blk, out_dtype=jnp.float32):
    m, k = x.shape
    n = w.shape[1]
    return pl.pallas_call(
        _mm_bias_kernel,
        grid=(m // blk,),
        in_specs=[
            pl.BlockSpec((blk, k), lambda i: (i, 0)),
            pl.BlockSpec((k, n), lambda i: (0, 0)),
            pl.BlockSpec((1, n), lambda i: (0, 0)),
        ],
        out_specs=pl.BlockSpec((blk, n), lambda i: (i, 0)),
        out_shape=jax.ShapeDtypeStruct((m, n), out_dtype),
    )(x, w, b.reshape(1, n))


def _taps_kernel(q_ref, woff_ref, boff_ref, wattn_ref, battn_ref, g_ref,
                 refx_ref, refy_ref, fconst_ref, iconst_ref,
                 p0_ref, p1_ref, p2_ref, p3_ref):
    q = q_ref[0]
    off2 = jnp.dot(q, woff_ref[...], preferred_element_type=jnp.float32) + boff_ref[...]
    z = jnp.dot(q, wattn_ref[...], preferred_element_type=jnp.float32) + battn_ref[...]
    e = jnp.exp(z)
    s = jnp.dot(e, g_ref[...], preferred_element_type=jnp.float32)
    aw = e / s

    invw = fconst_ref[0:1]
    invh = fconst_ref[1:2]
    wf = fconst_ref[2:3]
    hf = fconst_ref[3:4]
    wi = iconst_ref[0:1]
    hi = iconst_ref[1:2]
    n = pl.program_id(0)
    basev = iconst_ref[pl.ds(2 + n, 1)]

    px = (refx_ref[0] + off2[:, :HLP] * invw) * wf - 0.5
    py = (refy_ref[0] + off2[:, HLP:] * invh) * hf - 0.5
    x0f = jnp.floor(px)
    y0f = jnp.floor(py)
    lx = px - x0f
    ly = py - y0f
    x0 = x0f.astype(jnp.int32)
    y0 = y0f.astype(jnp.int32)

    outs = ((p0_ref, 0, 0, (1.0 - lx) * (1.0 - ly)),
            (p1_ref, 0, 1, lx * (1.0 - ly)),
            (p2_ref, 1, 0, (1.0 - lx) * ly),
            (p3_ref, 1, 1, lx * ly))
    for p_ref, dy, dx, bl in outs:
        xi = x0 + dx
        yi = y0 + dy
        valid = (xi >= 0) & (xi < wi) & (yi >= 0) & (yi < hi)
        xc = jnp.clip(xi, 0, wi - 1)
        yc = jnp.clip(yi, 0, hi - 1)
        idx = basev + (yc * wi + xc) * HEADS
        wt = aw * bl * valid.astype(jnp.float32)
        q = jnp.minimum((wt * 32768.0).astype(jnp.int32), 32767)
        p_ref[0] = (idx << 15) | q


def _compute_taps(q3, woffp, boffp, wattn, battn, g, refx, refy, fconst, iconst, bq):
    n, lq, _ = q3.shape
    grid = (n, lq // bq)
    row_specs = pl.BlockSpec((1, bq, HLP), lambda i, j: (i, j, 0))
    out_sd = jax.ShapeDtypeStruct((n, lq, HLP), jnp.int32)
    return pl.pallas_call(
        _taps_kernel,
        grid=grid,
        in_specs=[
            pl.BlockSpec((1, bq, EMBED), lambda i, j: (i, j, 0)),
            pl.BlockSpec((EMBED, 2 * HLP), lambda i, j: (0, 0)),
            pl.BlockSpec((1, 2 * HLP), lambda i, j: (0, 0)),
            pl.BlockSpec((EMBED, HLP), lambda i, j: (0, 0)),
            pl.BlockSpec((1, HLP), lambda i, j: (0, 0)),
            pl.BlockSpec((HLP, HLP), lambda i, j: (0, 0)),
            row_specs,
            row_specs,
            pl.BlockSpec((8, HLP), lambda i, j: (0, 0)),
            pl.BlockSpec((8, HLP), lambda i, j: (0, 0)),
        ],
        out_specs=(row_specs,) * 4,
        out_shape=(out_sd,) * 4,
    )(q3, woffp, boffp.reshape(1, 2 * HLP), wattn, battn.reshape(1, HLP),
      g, refx, refy, fconst, iconst)


SC_CORES = 2
SC_SUBCORES = 16


def _make_sc_gather(s_total, table_rows):
    nw = SC_CORES * SC_SUBCORES
    chunk = 16
    per_w = s_total // nw
    n_chunks = per_w // chunk
    kb = chunk * LEVELS * POINTS
    kh = kb // 2

    mesh = plsc.VectorSubcoreMesh(core_axis_name="c", subcore_axis_name="s",
                                  num_cores=SC_CORES, num_subcores=SC_SUBCORES)

    scratch = (
        [pltpu.VMEM((kb,), jnp.int32) for _ in range(8)]
        + [pltpu.VMEM((kb,), jnp.int32) for _ in range(8)]
        + [pltpu.VMEM((kb, DH // 2), jnp.int32) for _ in range(8)]
        + [pltpu.VMEM((chunk * DH,), jnp.float32) for _ in range(2)]
        + [pltpu.SemaphoreType.DMA for _ in range(6)]
    )

    @functools.partial(
        pl.kernel,
        out_type=jax.ShapeDtypeStruct((s_total * DH,), jnp.float32),
        mesh=mesh,
        scratch_types=scratch,
        compiler_params=pltpu.CompilerParams(use_tc_tiling_on_sc=False,
                                             needs_layout_passes=False),
    )
    def sc_gather(table, p0, p1, p2, p3, out, *scr):
        pv = (scr[0:4], scr[4:8])
        iv = (scr[8:12], scr[12:16])
        rv = (scr[16:20], scr[20:24])
        ov = scr[24:26]
        sem_i = scr[26:28]
        sem_r = scr[28:30]
        sem_o = scr[30:32]
        pk_hbm = (p0, p1, p2, p3)
        wid = lax.axis_index("s") * SC_CORES + lax.axis_index("c")
        w_base = wid * per_w
        lp = LEVELS * POINTS

        def fire_idx(c, s):
            fb = (w_base + c * chunk) * lp
            for t in range(4):
                pltpu.async_copy(pk_hbm[t].at[pl.ds(fb, kb)], pv[s][t], sem_i[s])

        def drain_idx(s):
            for t in range(4):
                pltpu.make_async_copy(pk_hbm[t].at[pl.ds(0, kb)], pv[s][t], sem_i[s]).wait()

        def unpack_idx(s):
            for t in range(4):
                for r in range(kb // 16):
                    v = pv[s][t][pl.ds(r * 16, 16)]
                    iv[s][t][pl.ds(r * 16, 16)] = lax.shift_right_logical(v, 15)

        def fire_rows(s):
            for t in range(4):
                for h in range(2):
                    pltpu.async_copy(table.at[iv[s][t].at[pl.ds(h * kh, kh)]],
                                     rv[s][t].at[pl.ds(h * kh, kh)], sem_r[s])

        def drain_rows(s):
            for t in range(4):
                for h in range(2):
                    pltpu.make_async_copy(
                        table.at[pl.ds(0, kh)],
                        rv[s][t].at[pl.ds(h * kh, kh)], sem_r[s]).wait()

        def fire_out(c, s):
            fb = (w_base + c * chunk) * DH
            pltpu.async_copy(ov[s], out.at[pl.ds(fb, chunk * DH)], sem_o[s])

        def drain_out(s):
            pltpu.make_async_copy(ov[s], out.at[pl.ds(0, chunk * DH)], sem_o[s]).wait()

        fire_idx(0, 0)
        fire_idx(1, 1)
        drain_idx(0)
        unpack_idx(0)
        fire_rows(0)

        def chunk_body(c, carry):
            for s in range(2):
                i = c * 2 + s
                drain_rows(s)

                @pl.when(i + 1 < n_chunks)
                def _():
                    drain_idx(1 - s)
                    unpack_idx(1 - s)
                    fire_rows(1 - s)

                @pl.when(i >= 2)
                def _():
                    drain_out(s)

                def sample_body(k0, carry2):
                    accs = [jnp.zeros((16,), jnp.float32) for _ in range(DH // 16)]
                    for t in range(4):
                        pk = pv[s][t][pl.ds(k0 * lp, lp)]
                        wvec = (pk & 32767).astype(jnp.float32) * (1.0 / 32768.0)
                        for j in range(lp):
                            k = k0 * lp + j
                            wt = wvec[j]
                            for g in range(DH // 32):
                                iv32 = rv[s][t][k, pl.ds(g * 16, 16)]
                                ea = plsc.bitcast(iv32 << 16, jnp.float32)
                                eb = plsc.bitcast(iv32, jnp.float32)
                                accs[2 * g] = accs[2 * g] + wt * ea
                                accs[2 * g + 1] = accs[2 * g + 1] + wt * eb
                    for d in range(DH // 16):
                        ov[s][pl.ds(k0 * DH + d * 16, 16)] = accs[d]
                    return carry2

                lax.fori_loop(0, chunk, sample_body, 0)
                fire_out(i, s)

                @pl.when(i + 2 < n_chunks)
                def _():
                    fire_idx(i + 2, s)
            return carry

        lax.fori_loop(0, n_chunks // 2, chunk_body, 0)
        drain_out(0)
        drain_out(1)

    return sc_gather


def kernel(query, key, value, reference_points, spatial_shapes,
           level_start_index, W_off, b_off, W_attn, b_attn, W_val, b_val,
           W_out, b_out):
    lq, n, c = query.shape
    lin = value.shape[0]
    s_total = n * lq * HEADS

    q3 = query.transpose(1, 0, 2)
    v2d = value.transpose(1, 0, 2).reshape(n * lin, c)

    perm_v = np.concatenate([np.arange(HEADS)[:, None] * DH + np.arange(32)[None, :],
                             np.arange(HEADS)[:, None] * DH + 32 + np.arange(32)[None, :]]
                            ).reshape(2, HEADS * 32).reshape(-1)
    w_val2 = W_val[:, jnp.array(perm_v)]
    b_val2 = b_val[jnp.array(perm_v)]
    vproj = _matmul_pack(v2d, w_val2, b_val2, blk=640)
    table = vproj.reshape(n * lin * HEADS, DH // 2)

    wf = jnp.tile(jnp.repeat(spatial_shapes[:, 1], POINTS), HEADS).astype(jnp.float32)
    hf = jnp.tile(jnp.repeat(spatial_shapes[:, 0], POINTS), HEADS).astype(jnp.float32)
    fconst = jnp.zeros((8, HLP), jnp.float32)
    fconst = fconst.at[0].set(1.0 / wf).at[1].set(1.0 / hf).at[2].set(wf).at[3].set(hf)
    wi = jnp.tile(jnp.repeat(spatial_shapes[:, 1], POINTS), HEADS)
    hi = jnp.tile(jnp.repeat(spatial_shapes[:, 0], POINTS), HEADS)
    start_lane = jnp.tile(jnp.repeat(level_start_index, POINTS), HEADS)
    h_lane = jnp.repeat(jnp.arange(HEADS, dtype=jnp.int32), LEVELS * POINTS)
    iconst = jnp.zeros((8, HLP), jnp.int32)
    iconst = iconst.at[0].set(wi).at[1].set(hi)
    for nn in range(n):
        iconst = iconst.at[2 + nn].set((nn * lin + start_lane) * HEADS + h_lane)

    woffp = W_off.reshape(c, HLP, 2).transpose(0, 2, 1).reshape(c, 2 * HLP)
    boffp = b_off.reshape(HLP, 2).T.reshape(2 * HLP)

    g = jnp.kron(jnp.eye(HEADS * LEVELS, dtype=jnp.float32),
                 jnp.ones((POINTS, POINTS), jnp.float32))

    rp_x = reference_points[..., 0]
    rp_y = reference_points[..., 1]
    refx = jnp.broadcast_to(rp_x[:, :, None, :, None],
                            (n, lq, HEADS, LEVELS, POINTS)).reshape(n, lq, HLP)
    refy = jnp.broadcast_to(rp_y[:, :, None, :, None],
                            (n, lq, HEADS, LEVELS, POINTS)).reshape(n, lq, HLP)

    taps = _compute_taps(q3, woffp, boffp, W_attn, b_attn, g, refx, refy,
                         fconst, iconst, bq=320)
    pk_flat = [t.reshape(s_total * LEVELS * POINTS) for t in taps]

    sc = _make_sc_gather(s_total, table.shape[0])
    sampled = sc(table, *pk_flat)
    sampled2 = sampled.reshape(n * lq, HEADS * DH)

    order64 = np.concatenate([np.arange(0, 16), np.arange(32, 48),
                              np.arange(16, 32), np.arange(48, 64)])
    perm = (np.arange(HEADS)[:, None] * DH + order64[None, :]).reshape(-1)
    out = _matmul_bias(sampled2, W_out[jnp.array(perm)], b_out, blk=640)
    return out.reshape(n, lq, c).transpose(1, 0, 2)
